# baseline (speedup 1.0000x reference)
import functools

import jax
import jax.numpy as jnp
from jax import lax
from jax.experimental import pallas as pl
from jax.experimental.pallas import tpu as pltpu
from jax.experimental.pallas import tpu_sc as plsc

_NSEQ = 16
_D = 512
_MAXLEN = 4096
_STEP = 256
_NWORK = 32
_ROWS_PER_W = _NSEQ * _MAXLEN // _NWORK
_CHUNK = 64


def _pad_body(*refs):
    seqs = refs[:_NSEQ]
    out = refs[_NSEQ]
    buf = refs[_NSEQ + 1]
    zbuf = refs[_NSEQ + 2]

    zero16 = jnp.zeros((16,), jnp.float32)

    def _zr(r, c):
        def _zc(ci, c2):
            zbuf[r, pl.ds(ci * 16, 16)] = zero16
            return c2
        return lax.fori_loop(0, _D // 16, _zc, c)

    lax.fori_loop(0, _CHUNK, _zr, 0)

    wid = lax.axis_index("s") * 2 + lax.axis_index("c")
    i = wid // 2
    h = wid % 2
    row0 = h * _ROWS_PER_W
    length = _MAXLEN - _STEP * i
    n_copy = pl.multiple_of(jnp.clip(length - row0, 0, _ROWS_PER_W), _CHUNK)
    dst0 = pl.multiple_of(i * _MAXLEN + row0, _ROWS_PER_W)

    for k in range(_NSEQ):
        @pl.when(i == k)
        def _(k=k):
            src = seqs[k]

            def _cp(j, c):
                off = pl.multiple_of(j * _CHUNK, _CHUNK)
                pltpu.sync_copy(src.at[pl.ds(row0 + off, _CHUNK)], buf)
                pltpu.sync_copy(buf, out.at[pl.ds(dst0 + off, _CHUNK)])
                return c

            lax.fori_loop(0, n_copy // _CHUNK, _cp, 0)

    def _zf(j, c):
        zoff = pl.multiple_of(dst0 + n_copy + j * _CHUNK, _CHUNK)
        pltpu.sync_copy(zbuf, out.at[pl.ds(zoff, _CHUNK)])
        return c

    lax.fori_loop(0, (_ROWS_PER_W - n_copy) // _CHUNK, _zf, 0)


_pad_call = pl.kernel(
    _pad_body,
    out_type=jax.ShapeDtypeStruct((_NSEQ * _MAXLEN, _D), jnp.float32),
    mesh=plsc.VectorSubcoreMesh(core_axis_name="c", subcore_axis_name="s"),
    scratch_types=[
        pltpu.VMEM((_CHUNK, _D), jnp.float32),
        pltpu.VMEM((_CHUNK, _D), jnp.float32),
    ],
)


def _mask_body(o_ref):
    col = lax.broadcasted_iota(jnp.int32, (_NSEQ, _MAXLEN), 1)
    row = lax.broadcasted_iota(jnp.int32, (_NSEQ, _MAXLEN), 0)
    o_ref[...] = col >= (_MAXLEN - _STEP * row)


_mask_call = pl.pallas_call(
    _mask_body,
    out_shape=jax.ShapeDtypeStruct((_NSEQ, _MAXLEN), jnp.bool_),
)


@jax.jit
def kernel(seq_0, seq_1, seq_2, seq_3, seq_4, seq_5, seq_6, seq_7, seq_8,
           seq_9, seq_10, seq_11, seq_12, seq_13, seq_14, seq_15):
    seqs = (seq_0, seq_1, seq_2, seq_3, seq_4, seq_5, seq_6, seq_7, seq_8,
            seq_9, seq_10, seq_11, seq_12, seq_13, seq_14, seq_15)
    padded = _pad_call(*seqs).reshape(_NSEQ, _MAXLEN, _D)
    padding_mask = _mask_call()
    return padded, padding_mask
